# baseline (speedup 1.0000x reference)
import functools

import jax
import jax.numpy as jnp
import numpy as np
from jax import lax
from jax.experimental import pallas as pl
from jax.experimental.pallas import tpu as pltpu
from jax.experimental.pallas import tpu_sc as plsc

N = 10000
E = 320000
D_IN = 128
D_H = 128
D_OUT = 64

NC = 2
NS = 16
L = 16

_MESH = plsc.VectorSubcoreMesh(core_axis_name="c", subcore_axis_name="s")


_EPW = E // (NC * NS)
_DEG_CHUNKS = _EPW // 128
_DEG_TAIL = _EPW - _DEG_CHUNKS * 128
_NPAD = 10240


def _deg_body(row_hbm, out_hbm, hist, idxbuf, obuf, shared):
    c = lax.axis_index("c")
    s = lax.axis_index("s")
    base = (c * NS + s) * _EPW

    def _z(i, _):
        hist[pl.ds(i * L, L)] = jnp.zeros((L,), jnp.float32)
        return 0
    lax.fori_loop(0, _NPAD // L, _z, 0)

    ones = jnp.ones((L,), jnp.float32)

    def _chunk(k, _):
        pltpu.sync_copy(row_hbm.at[pl.ds(base + k * 128, 128)], idxbuf)
        def _inner(j, _):
            plsc.addupdate_scatter(hist, [idxbuf[pl.ds(j * L, L)]], ones)
            return 0
        lax.fori_loop(0, 128 // L, _inner, 0)
        return 0
    lax.fori_loop(0, _DEG_CHUNKS, _chunk, 0)

    pltpu.sync_copy(
        row_hbm.at[pl.ds(base + _DEG_CHUNKS * 128, _DEG_TAIL)],
        idxbuf.at[pl.ds(0, _DEG_TAIL)])
    plsc.addupdate_scatter(hist, [idxbuf[pl.ds(0, _DEG_TAIL)]], ones)

    pltpu.sync_copy(hist, shared.at[pl.ds(s * _NPAD, _NPAD)])
    plsc.subcore_barrier()

    for t in range(NS):
        pltpu.sync_copy(shared.at[pl.ds(t * _NPAD + s * 640, 640)],
                        hist.at[pl.ds(t * 640, 640)])

    def _red(j, _):
        vacc = jnp.zeros((L,), jnp.float32)
        for t in range(NS):
            vacc = vacc + hist[pl.ds(t * 640 + j * L, L)]
        obuf[pl.ds(j * L, L)] = vacc
        return 0
    lax.fori_loop(0, 640 // L, _red, 0)

    pltpu.sync_copy(obuf, out_hbm.at[pl.ds(c * _NPAD + s * 640, 640)])


_deg_call = pl.kernel(
    _deg_body,
    out_type=jax.ShapeDtypeStruct((NC * _NPAD,), jnp.float32),
    mesh=_MESH,
    scratch_types=[
        pltpu.VMEM((_NPAD,), jnp.float32),
        pltpu.VMEM((128,), jnp.int32),
        pltpu.VMEM((640,), jnp.float32),
        pltpu.VMEM_SHARED((NS * _NPAD,), jnp.float32),
    ],
    compiler_params=pltpu.CompilerParams(needs_layout_passes=False),
    name="sc_degree",
)


_SUP = 13
_NSUPER = 192
_NCHUNK = E // 128
_NSUP1 = 12
_NSUP2 = 6
_RPT = 624


def _spmm_body(nsup, split_edges,
               t_hbm, row_hbm, col_hbm, row3d, col3d, z_hbm, s_hbm,
               colsb, rowsb, colbuf, rowbuf, gbuf2,
               semg0, semg1, semr0, semr1, semc0, semc1,
               sems0, sems1, sem, acc):
    c = lax.axis_index("c")
    s = lax.axis_index("s")

    pltpu.sync_copy(z_hbm.at[pl.ds(0, _RPT)], acc.at[pl.ds(s * _RPT, _RPT)])

    @pl.when(s == NS - 1)
    def _():
        pltpu.sync_copy(z_hbm.at[pl.ds(0, 16)], acc.at[pl.ds(NS * _RPT, 16)])

    if split_edges:
        gbase = 96 * c + _NSUP2 * s
    else:
        gbase = _NSUP1 * s
    offv = jnp.full((L,), c * N, jnp.int32) if not split_edges else None

    plsc.subcore_barrier()

    sems_g = (semg0, semg1)
    sems_r = (semr0, semr1)
    sems_c = (semc0, semc1)
    sems_s = (sems0, sems1)

    def _fetch(u, sb):
        g = gbase + u
        pltpu.async_copy(row3d.at[g], rowsb.at[pl.ds(sb * _SUP, _SUP)],
                         sems_r[sb])
        pltpu.async_copy(col3d.at[g], colsb.at[pl.ds(sb * _SUP, _SUP)],
                         sems_c[sb])

    def _wait_adjust(sb):
        pltpu.make_async_copy(row3d.at[0], rowsb.at[pl.ds(sb * _SUP, _SUP)],
                              sems_r[sb]).wait()
        pltpu.make_async_copy(col3d.at[0], colsb.at[pl.ds(sb * _SUP, _SUP)],
                              sems_c[sb]).wait()
        if offv is not None:
            for j in range(_SUP):
                for jb in range(128 // L):
                    colsb[sb * _SUP + j, pl.ds(jb * L, L)] = (
                        colsb[sb * _SUP + j, pl.ds(jb * L, L)] + offv)

    def _g_issue(sb, j, b):
        pltpu.async_copy(t_hbm.at[colsb.at[sb * _SUP + j]], gbuf2.at[b],
                         sems_g[b])

    def _s_wait(bb):
        pltpu.make_async_copy(gbuf2.at[bb], acc.at[rowsb.at[0]],
                              sems_s[bb]).wait()

    _fetch(0, 0)
    _fetch(1, 1)
    _wait_adjust(0)
    _g_issue(0, 0, 0)

    def _pair(p, _):
        for up in (0, 1):
            u = p * 2 + up
            sb = up
            for j in range(_SUP):
                b = (up + j) % 2
                pltpu.make_async_copy(t_hbm.at[pl.ds(0, 128)],
                                      gbuf2.at[b], sems_g[b]).wait()
                nb = 1 - b
                if j < _SUP - 1:
                    if up == 0 and j == 0:
                        @pl.when(p > 0)
                        def _():
                            _s_wait(nb)
                    else:
                        _s_wait(nb)
                    _g_issue(sb, j + 1, nb)
                else:
                    @pl.when(u + 1 < nsup)
                    def _():
                        _wait_adjust(1 - sb)
                        _s_wait(nb)
                        _g_issue(1 - sb, 0, nb)
                pltpu.async_copy(gbuf2.at[b],
                                 acc.at[rowsb.at[sb * _SUP + j]],
                                 sems_s[b], add=True)
            @pl.when(u + 2 < nsup)
            def _():
                _fetch(u + 2, sb)
        return 0
    lax.fori_loop(0, nsup // 2, _pair, 0)

    _s_wait(0)
    _s_wait(1)

    if split_edges:
        nex = 2
        exrow = _NSUPER * _SUP + 2 * c + s
    else:
        nex = 4
        exrow = _NSUPER * _SUP + s

    @pl.when(s < nex)
    def _():
        eex = exrow * 128
        pltpu.sync_copy(row_hbm.at[pl.ds(eex, 128)], rowbuf)
        pltpu.sync_copy(col_hbm.at[pl.ds(eex, 128)], colbuf)
        if offv is not None:
            for jb in range(128 // L):
                colbuf[pl.ds(jb * L, L)] = colbuf[pl.ds(jb * L, L)] + offv
        pltpu.async_copy(t_hbm.at[colbuf], gbuf2.at[0], sem).wait()
        pltpu.sync_copy(gbuf2.at[0], acc.at[rowbuf], add=True)

    plsc.subcore_barrier()

    obase = c * N + s * _RPT
    pltpu.sync_copy(acc.at[pl.ds(s * _RPT, _RPT)],
                    s_hbm.at[pl.ds(obase, _RPT)])

    @pl.when(s == NS - 1)
    def _():
        pltpu.sync_copy(acc.at[pl.ds(NS * _RPT, 16)],
                        s_hbm.at[pl.ds(c * N + NS * _RPT, 16)])


def _spmm_scratch(d_feat):
    return [
        pltpu.VMEM((2 * _SUP, 128), jnp.int32),
        pltpu.VMEM((2 * _SUP, 128), jnp.int32),
        pltpu.VMEM((128,), jnp.int32),
        pltpu.VMEM((128,), jnp.int32),
        pltpu.VMEM((2, 128, d_feat), jnp.float32),
        pltpu.SemaphoreType.DMA,
        pltpu.SemaphoreType.DMA,
        pltpu.SemaphoreType.DMA,
        pltpu.SemaphoreType.DMA,
        pltpu.SemaphoreType.DMA,
        pltpu.SemaphoreType.DMA,
        pltpu.SemaphoreType.DMA,
        pltpu.SemaphoreType.DMA,
        pltpu.SemaphoreType.DMA,
        pltpu.VMEM_SHARED((N, d_feat), jnp.float32),
    ]


_spmm128 = pl.kernel(
    functools.partial(_spmm_body, _NSUP1, False),
    out_type=jax.ShapeDtypeStruct((2 * N, D_H), jnp.float32),
    mesh=_MESH,
    scratch_types=_spmm_scratch(D_H),
    compiler_params=pltpu.CompilerParams(needs_layout_passes=False),
    name="sc_spmm_128",
)

_spmm2 = pl.kernel(
    functools.partial(_spmm_body, _NSUP2, True),
    out_type=jax.ShapeDtypeStruct((2 * N, 128), jnp.float32),
    mesh=_MESH,
    scratch_types=_spmm_scratch(128),
    compiler_params=pltpu.CompilerParams(needs_layout_passes=False),
    name="sc_spmm_packed64",
)


_R = 1000


def _elu(a):
    return jnp.where(a > 0, a, jnp.exp(a) - 1.0)


def _dense01_body(x_ref, p0_ref, p1_ref, wm0_ref, bm0_ref, wv0_ref, bv0_ref,
                  wm1_ref, bm1_ref, wv1_ref, bv1_ref,
                  hb_ref, d0_ref, d1_ref):
    deg = p0_ref[...] + p1_ref[...]
    d0 = jnp.where(deg > 0, lax.rsqrt(deg), 0.0)
    d1 = d0 * d0
    x = x_ref[...]
    mean = _elu(jnp.dot(x, wm0_ref[...],
                        preferred_element_type=jnp.float32) + bm0_ref[...])
    var = jnp.maximum(jnp.dot(x, wv0_ref[...],
                              preferred_element_type=jnp.float32)
                      + bv0_ref[...], 0.0)
    m = _elu(jnp.dot(mean, wm1_ref[...],
                     preferred_element_type=jnp.float32) + bm1_ref[...])
    v = jnp.maximum(jnp.dot(var, wv1_ref[...],
                            preferred_element_type=jnp.float32)
                    + bv1_ref[...], 0.0) + 1e-6
    att = jnp.exp(-v)
    hb_ref[0] = d0 * (m * att)
    hb_ref[1] = d1 * (v * (att * att))
    d0_ref[...] = d0
    d1_ref[...] = d1


def _dense2_body(sm_ref, sv_ref, d0_ref, d1_ref, wm2_ref, bm2_ref,
                 wv2_ref, bv2_ref, hb_ref):
    d0 = d0_ref[...]
    d1 = d1_ref[...]
    mean = d0 * sm_ref[...]
    var = d1 * sv_ref[...]
    m = _elu(jnp.dot(mean, wm2_ref[...],
                     preferred_element_type=jnp.float32) + bm2_ref[...])
    v = jnp.maximum(jnp.dot(var, wv2_ref[...],
                            preferred_element_type=jnp.float32)
                    + bv2_ref[...], 0.0) + 1e-6
    att = jnp.exp(-v)
    hb_ref[...] = jnp.concatenate(
        [d0 * (m * att), d1 * (v * (att * att))], axis=1)


def _final_body(s0_ref, s1_ref, d0_ref, d1_ref, smp_ref, out_ref):
    tot = s0_ref[...] + s1_ref[...]
    mean = d0_ref[...] * tot[:, :D_OUT]
    var = d1_ref[...] * tot[:, D_OUT:]
    out = mean + smp_ref[...] * jnp.sqrt(jnp.clip(var, 1e-12, None))
    out = out - jnp.max(out, axis=1, keepdims=True)
    out_ref[...] = out - jnp.log(
        jnp.sum(jnp.exp(out), axis=1, keepdims=True))


def _row_spec(w):
    return pl.BlockSpec((_R, w), lambda i: (i, 0))


def _full_spec(shape):
    return pl.BlockSpec(shape, lambda i: tuple(0 for _ in shape))


_dense01_call = pl.pallas_call(
    _dense01_body,
    grid=(N // _R,),
    in_specs=[
        _row_spec(D_IN), _row_spec(1), _row_spec(1),
        _full_spec((D_IN, D_H)), _full_spec((1, D_H)),
        _full_spec((D_IN, D_H)), _full_spec((1, D_H)),
        _full_spec((D_H, D_H)), _full_spec((1, D_H)),
        _full_spec((D_H, D_H)), _full_spec((1, D_H)),
    ],
    out_specs=[
        pl.BlockSpec((2, _R, D_H), lambda i: (0, i, 0)),
        _row_spec(1), _row_spec(1),
    ],
    out_shape=[
        jax.ShapeDtypeStruct((2, N, D_H), jnp.float32),
        jax.ShapeDtypeStruct((N, 1), jnp.float32),
        jax.ShapeDtypeStruct((N, 1), jnp.float32),
    ],
    name="tc_dense01",
)

_dense2_call = pl.pallas_call(
    _dense2_body,
    grid=(N // _R,),
    in_specs=[
        pl.BlockSpec((_R, D_H), lambda i: (i, 0)),
        pl.BlockSpec((_R, D_H), lambda i: (N // _R + i, 0)),
        _row_spec(1), _row_spec(1),
        _full_spec((D_H, D_OUT)), _full_spec((1, D_OUT)),
        _full_spec((D_H, D_OUT)), _full_spec((1, D_OUT)),
    ],
    out_specs=[_row_spec(2 * D_OUT)],
    out_shape=[jax.ShapeDtypeStruct((N, 2 * D_OUT), jnp.float32)],
    name="tc_dense2",
)

_final_call = pl.pallas_call(
    _final_body,
    grid=(N // _R,),
    in_specs=[
        pl.BlockSpec((_R, 2 * D_OUT), lambda i: (i, 0)),
        pl.BlockSpec((_R, 2 * D_OUT), lambda i: (N // _R + i, 0)),
        _row_spec(1), _row_spec(1),
        _row_spec(D_OUT),
    ],
    out_specs=[_row_spec(D_OUT)],
    out_shape=[jax.ShapeDtypeStruct((N, D_OUT), jnp.float32)],
    name="tc_final",
)


def kernel(x, edge_index, Wm0, bm0, Wv0, bv0, Wm1, bm1, Wv1, bv1,
           Wm2, bm2, Wv2, bv2):
    row = edge_index[0]
    col = edge_index[1]

    parts = _deg_call(row)
    pp = parts.reshape(NC, _NPAD)
    p0 = pp[0, :N].reshape(N, 1)
    p1 = pp[1, :N].reshape(N, 1)

    hb, d0, d1 = _dense01_call(
        x, p0, p1,
        Wm0, bm0.reshape(1, D_H), Wv0, bv0.reshape(1, D_H),
        Wm1, bm1.reshape(1, D_H), Wv1, bv1.reshape(1, D_H))

    zrows = jnp.zeros((640, 128), jnp.float32)
    row3d = row[:_NSUPER * _SUP * 128].reshape(_NSUPER, _SUP, 128)
    col3d = col[:_NSUPER * _SUP * 128].reshape(_NSUPER, _SUP, 128)
    s1 = _spmm128(hb.reshape(2 * N, D_H), row, col, row3d, col3d,
                  zrows)

    (hb2,) = _dense2_call(
        s1, s1, d0, d1,
        Wm2, bm2.reshape(1, D_OUT), Wv2, bv2.reshape(1, D_OUT))

    s2 = _spmm2(hb2, row, col, row3d, col3d, zrows)

    sample = jax.random.normal(jax.random.key(42), (N, D_OUT),
                               dtype=jnp.float32)
    (out,) = _final_call(s2, s2, d0, d1, sample)
    return out


# speedup vs baseline: 1.2793x; 1.2793x over previous
import functools

import jax
import jax.numpy as jnp
import numpy as np
from jax import lax
from jax.experimental import pallas as pl
from jax.experimental.pallas import tpu as pltpu
from jax.experimental.pallas import tpu_sc as plsc

N = 10000
E = 320000
D_IN = 128
D_H = 128
D_OUT = 64

NC = 2
NS = 16
L = 16

_MESH = plsc.VectorSubcoreMesh(core_axis_name="c", subcore_axis_name="s")


_EPW = E // (NC * NS)
_DEG_CHUNKS = _EPW // 128
_DEG_TAIL = _EPW - _DEG_CHUNKS * 128
_NPAD = 10240


def _deg_body(row_hbm, out_hbm, hist, idxbuf, obuf, shared):
    c = lax.axis_index("c")
    s = lax.axis_index("s")
    base = (c * NS + s) * _EPW

    def _z(i, _):
        hist[pl.ds(i * L, L)] = jnp.zeros((L,), jnp.float32)
        return 0
    lax.fori_loop(0, _NPAD // L, _z, 0)

    ones = jnp.ones((L,), jnp.float32)

    def _chunk(k, _):
        pltpu.sync_copy(row_hbm.at[pl.ds(base + k * 128, 128)], idxbuf)
        def _inner(j, _):
            plsc.addupdate_scatter(hist, [idxbuf[pl.ds(j * L, L)]], ones)
            return 0
        lax.fori_loop(0, 128 // L, _inner, 0)
        return 0
    lax.fori_loop(0, _DEG_CHUNKS, _chunk, 0)

    pltpu.sync_copy(
        row_hbm.at[pl.ds(base + _DEG_CHUNKS * 128, _DEG_TAIL)],
        idxbuf.at[pl.ds(0, _DEG_TAIL)])
    plsc.addupdate_scatter(hist, [idxbuf[pl.ds(0, _DEG_TAIL)]], ones)

    pltpu.sync_copy(hist, shared.at[pl.ds(s * _NPAD, _NPAD)])
    plsc.subcore_barrier()

    for t in range(NS):
        pltpu.sync_copy(shared.at[pl.ds(t * _NPAD + s * 640, 640)],
                        hist.at[pl.ds(t * 640, 640)])

    def _red(j, _):
        vacc = jnp.zeros((L,), jnp.float32)
        for t in range(NS):
            vacc = vacc + hist[pl.ds(t * 640 + j * L, L)]
        obuf[pl.ds(j * L, L)] = vacc
        return 0
    lax.fori_loop(0, 640 // L, _red, 0)

    pltpu.sync_copy(obuf, out_hbm.at[pl.ds(c * _NPAD + s * 640, 640)])


_deg_call = pl.kernel(
    _deg_body,
    out_type=jax.ShapeDtypeStruct((NC * _NPAD,), jnp.float32),
    mesh=_MESH,
    scratch_types=[
        pltpu.VMEM((_NPAD,), jnp.float32),
        pltpu.VMEM((128,), jnp.int32),
        pltpu.VMEM((640,), jnp.float32),
        pltpu.VMEM_SHARED((NS * _NPAD,), jnp.float32),
    ],
    compiler_params=pltpu.CompilerParams(needs_layout_passes=False),
    name="sc_degree",
)


_SUP = 13
_NSUPER = 192
_NCHUNK = E // 128
_NSUP1 = 12
_NSUP2 = 6
_RPT = 624


def _spmm_body(nsup, split_edges,
               t_hbm, row_hbm, col_hbm, row3d, col3d, z_hbm, s_hbm,
               colsb, rowsb, colbuf, rowbuf, gbuf2,
               semg0, semg1, semr0, semr1, semc0, semc1,
               sems0, sems1, sem, acc):
    c = lax.axis_index("c")
    s = lax.axis_index("s")

    pltpu.sync_copy(z_hbm.at[pl.ds(0, _RPT)], acc.at[pl.ds(s * _RPT, _RPT)])

    @pl.when(s == NS - 1)
    def _():
        pltpu.sync_copy(z_hbm.at[pl.ds(0, 16)], acc.at[pl.ds(NS * _RPT, 16)])

    if split_edges:
        gbase = 96 * c + _NSUP2 * s
    else:
        gbase = _NSUP1 * s
    offv = jnp.full((L,), c * N, jnp.int32) if not split_edges else None

    plsc.subcore_barrier()

    sems_g = (semg0, semg1)
    sems_r = (semr0, semr1)
    sems_c = (semc0, semc1)
    sems_s = (sems0, sems1)

    def _fetch(u, sb):
        g = gbase + u
        pltpu.async_copy(row3d.at[g], rowsb.at[pl.ds(sb * _SUP, _SUP)],
                         sems_r[sb])
        pltpu.async_copy(col3d.at[g], colsb.at[pl.ds(sb * _SUP, _SUP)],
                         sems_c[sb])

    def _wait_adjust(sb):
        pltpu.make_async_copy(row3d.at[0], rowsb.at[pl.ds(sb * _SUP, _SUP)],
                              sems_r[sb]).wait()
        pltpu.make_async_copy(col3d.at[0], colsb.at[pl.ds(sb * _SUP, _SUP)],
                              sems_c[sb]).wait()
        if offv is not None:
            for j in range(_SUP):
                for jb in range(128 // L):
                    colsb[sb * _SUP + j, pl.ds(jb * L, L)] = (
                        colsb[sb * _SUP + j, pl.ds(jb * L, L)] + offv)

    def _g_issue(sb, j, b):
        pltpu.async_copy(t_hbm.at[colsb.at[sb * _SUP + j]], gbuf2.at[b],
                         sems_g[b])

    def _s_wait(bb):
        pltpu.make_async_copy(gbuf2.at[bb], acc.at[rowsb.at[0]],
                              sems_s[bb]).wait()

    _fetch(0, 0)
    _fetch(1, 1)
    _wait_adjust(0)
    _g_issue(0, 0, 0)

    def _pair(p, _):
        for up in (0, 1):
            u = p * 2 + up
            sb = up
            for j in range(_SUP):
                b = (up + j) % 2
                pltpu.make_async_copy(t_hbm.at[pl.ds(0, 128)],
                                      gbuf2.at[b], sems_g[b]).wait()
                nb = 1 - b
                if j < _SUP - 1:
                    if up == 0 and j == 0:
                        @pl.when(p > 0)
                        def _():
                            _s_wait(nb)
                    else:
                        _s_wait(nb)
                    _g_issue(sb, j + 1, nb)
                else:
                    @pl.when(u + 1 < nsup)
                    def _():
                        _wait_adjust(1 - sb)
                        _s_wait(nb)
                        _g_issue(1 - sb, 0, nb)
                pltpu.async_copy(gbuf2.at[b],
                                 acc.at[rowsb.at[sb * _SUP + j]],
                                 sems_s[b], add=True)
            @pl.when(u + 2 < nsup)
            def _():
                _fetch(u + 2, sb)
        return 0
    lax.fori_loop(0, nsup // 2, _pair, 0)

    _s_wait(0)
    _s_wait(1)

    if split_edges:
        nex = 2
        exrow = _NSUPER * _SUP + 2 * c + s
    else:
        nex = 4
        exrow = _NSUPER * _SUP + s

    @pl.when(s < nex)
    def _():
        eex = exrow * 128
        pltpu.sync_copy(row_hbm.at[pl.ds(eex, 128)], rowbuf)
        pltpu.sync_copy(col_hbm.at[pl.ds(eex, 128)], colbuf)
        if offv is not None:
            for jb in range(128 // L):
                colbuf[pl.ds(jb * L, L)] = colbuf[pl.ds(jb * L, L)] + offv
        pltpu.async_copy(t_hbm.at[colbuf], gbuf2.at[0], sem).wait()
        pltpu.sync_copy(gbuf2.at[0], acc.at[rowbuf], add=True)

    plsc.subcore_barrier()

    obase = c * N + s * _RPT
    pltpu.sync_copy(acc.at[pl.ds(s * _RPT, _RPT)],
                    s_hbm.at[pl.ds(obase, _RPT)])

    @pl.when(s == NS - 1)
    def _():
        pltpu.sync_copy(acc.at[pl.ds(NS * _RPT, 16)],
                        s_hbm.at[pl.ds(c * N + NS * _RPT, 16)])


def _spmm_scratch(d_feat):
    return [
        pltpu.VMEM((2 * _SUP, 128), jnp.int32),
        pltpu.VMEM((2 * _SUP, 128), jnp.int32),
        pltpu.VMEM((128,), jnp.int32),
        pltpu.VMEM((128,), jnp.int32),
        pltpu.VMEM((2, 128, d_feat), jnp.float32),
        pltpu.SemaphoreType.DMA,
        pltpu.SemaphoreType.DMA,
        pltpu.SemaphoreType.DMA,
        pltpu.SemaphoreType.DMA,
        pltpu.SemaphoreType.DMA,
        pltpu.SemaphoreType.DMA,
        pltpu.SemaphoreType.DMA,
        pltpu.SemaphoreType.DMA,
        pltpu.SemaphoreType.DMA,
        pltpu.VMEM_SHARED((N, d_feat), jnp.float32),
    ]


_spmm2 = pl.kernel(
    functools.partial(_spmm_body, _NSUP2, True),
    out_type=jax.ShapeDtypeStruct((2 * N, 128), jnp.float32),
    mesh=_MESH,
    scratch_types=_spmm_scratch(128),
    compiler_params=pltpu.CompilerParams(needs_layout_passes=False),
    name="sc_spmm_packed64",
)


_R = 1000


def _elu(a):
    return jnp.where(a > 0, a, jnp.exp(a) - 1.0)


def _dense01_body(x_ref, p0_ref, p1_ref, wm0_ref, bm0_ref, wv0_ref, bv0_ref,
                  wm1_ref, bm1_ref, wv1_ref, bv1_ref, wm2_ref, wv2_ref,
                  hb_ref, d0_ref, d1_ref):
    deg = p0_ref[...] + p1_ref[...]
    d0 = jnp.where(deg > 0, lax.rsqrt(deg), 0.0)
    d1 = d0 * d0
    x = x_ref[...]
    mean = _elu(jnp.dot(x, wm0_ref[...],
                        preferred_element_type=jnp.float32) + bm0_ref[...])
    var = jnp.maximum(jnp.dot(x, wv0_ref[...],
                              preferred_element_type=jnp.float32)
                      + bv0_ref[...], 0.0)
    m = _elu(jnp.dot(mean, wm1_ref[...],
                     preferred_element_type=jnp.float32) + bm1_ref[...])
    v = jnp.maximum(jnp.dot(var, wv1_ref[...],
                            preferred_element_type=jnp.float32)
                    + bv1_ref[...], 0.0) + 1e-6
    att = jnp.exp(-v)
    tm = jnp.dot(m * att, wm2_ref[...], preferred_element_type=jnp.float32)
    tv = jnp.dot(v * (att * att), wv2_ref[...],
                 preferred_element_type=jnp.float32)
    hb_ref[...] = jnp.concatenate([d0 * tm, d1 * tv], axis=1)
    d0_ref[...] = d0
    d1_ref[...] = d1


def _dense2_body(s0_ref, s1_ref, d0_ref, d1_ref, bm2_ref, bv2_ref, hb_ref):
    tot = s0_ref[...] + s1_ref[...]
    d0 = d0_ref[...]
    d1 = d1_ref[...]
    m = _elu(d0 * tot[:, :D_OUT] + bm2_ref[...])
    v = jnp.maximum(d1 * tot[:, D_OUT:] + bv2_ref[...], 0.0) + 1e-6
    att = jnp.exp(-v)
    hb_ref[...] = jnp.concatenate(
        [d0 * (m * att), d1 * (v * (att * att))], axis=1)


def _final_body(s0_ref, s1_ref, d0_ref, d1_ref, smp_ref, out_ref):
    tot = s0_ref[...] + s1_ref[...]
    mean = d0_ref[...] * tot[:, :D_OUT]
    var = d1_ref[...] * tot[:, D_OUT:]
    out = mean + smp_ref[...] * jnp.sqrt(jnp.clip(var, 1e-12, None))
    out = out - jnp.max(out, axis=1, keepdims=True)
    out_ref[...] = out - jnp.log(
        jnp.sum(jnp.exp(out), axis=1, keepdims=True))


def _row_spec(w):
    return pl.BlockSpec((_R, w), lambda i: (i, 0))


def _full_spec(shape):
    return pl.BlockSpec(shape, lambda i: tuple(0 for _ in shape))


_dense01_call = pl.pallas_call(
    _dense01_body,
    grid=(N // _R,),
    in_specs=[
        _row_spec(D_IN), _row_spec(1), _row_spec(1),
        _full_spec((D_IN, D_H)), _full_spec((1, D_H)),
        _full_spec((D_IN, D_H)), _full_spec((1, D_H)),
        _full_spec((D_H, D_H)), _full_spec((1, D_H)),
        _full_spec((D_H, D_H)), _full_spec((1, D_H)),
        _full_spec((D_H, D_OUT)), _full_spec((D_H, D_OUT)),
    ],
    out_specs=[
        _row_spec(2 * D_OUT),
        _row_spec(1), _row_spec(1),
    ],
    out_shape=[
        jax.ShapeDtypeStruct((N, 2 * D_OUT), jnp.float32),
        jax.ShapeDtypeStruct((N, 1), jnp.float32),
        jax.ShapeDtypeStruct((N, 1), jnp.float32),
    ],
    name="tc_dense01",
)

_dense2_call = pl.pallas_call(
    _dense2_body,
    grid=(N // _R,),
    in_specs=[
        pl.BlockSpec((_R, 2 * D_OUT), lambda i: (i, 0)),
        pl.BlockSpec((_R, 2 * D_OUT), lambda i: (N // _R + i, 0)),
        _row_spec(1), _row_spec(1),
        _full_spec((1, D_OUT)), _full_spec((1, D_OUT)),
    ],
    out_specs=[_row_spec(2 * D_OUT)],
    out_shape=[jax.ShapeDtypeStruct((N, 2 * D_OUT), jnp.float32)],
    name="tc_dense2",
)

_final_call = pl.pallas_call(
    _final_body,
    grid=(N // _R,),
    in_specs=[
        pl.BlockSpec((_R, 2 * D_OUT), lambda i: (i, 0)),
        pl.BlockSpec((_R, 2 * D_OUT), lambda i: (N // _R + i, 0)),
        _row_spec(1), _row_spec(1),
        _row_spec(D_OUT),
    ],
    out_specs=[_row_spec(D_OUT)],
    out_shape=[jax.ShapeDtypeStruct((N, D_OUT), jnp.float32)],
    name="tc_final",
)


def kernel(x, edge_index, Wm0, bm0, Wv0, bv0, Wm1, bm1, Wv1, bv1,
           Wm2, bm2, Wv2, bv2):
    row = edge_index[0]
    col = edge_index[1]

    parts = _deg_call(row)
    pp = parts.reshape(NC, _NPAD)
    p0 = pp[0, :N].reshape(N, 1)
    p1 = pp[1, :N].reshape(N, 1)

    hb, d0, d1 = _dense01_call(
        x, p0, p1,
        Wm0, bm0.reshape(1, D_H), Wv0, bv0.reshape(1, D_H),
        Wm1, bm1.reshape(1, D_H), Wv1, bv1.reshape(1, D_H),
        Wm2, Wv2)

    zrows = jnp.zeros((640, 128), jnp.float32)
    row3d = row[:_NSUPER * _SUP * 128].reshape(_NSUPER, _SUP, 128)
    col3d = col[:_NSUPER * _SUP * 128].reshape(_NSUPER, _SUP, 128)
    s1 = _spmm2(hb, row, col, row3d, col3d, zrows)

    (hb2,) = _dense2_call(
        s1, s1, d0, d1,
        bm2.reshape(1, D_OUT), bv2.reshape(1, D_OUT))

    s2 = _spmm2(hb2, row, col, row3d, col3d, zrows)

    sample = jax.random.normal(jax.random.key(42), (N, D_OUT),
                               dtype=jnp.float32)
    (out,) = _final_call(s2, s2, d0, d1, sample)
    return out
